# baseline (speedup 1.0000x reference)
import functools
import jax
import jax.numpy as jnp
from jax import lax
from jax.experimental import pallas as pl
from jax.experimental.pallas import tpu as pltpu
from jax.experimental.pallas import tpu_sc as plsc

N = 10000
NPAD = 10240
E = 160000
NS = 16
NCORES = 2
EB = 64
NBLK = 160
EPT = NBLK * EB
EPAD = EPT * NS
ROWS_PT = NPAD // NS
BN = 256
GRID_N = NPAD // BN

_SC_PARAMS = pltpu.CompilerParams(use_tc_tiling_on_sc=False)
_MESH = plsc.VectorSubcoreMesh(core_axis_name="c", subcore_axis_name="s")


def _make_agg(nchunks, dc, dtype=jnp.float32):
    single_core = nchunks == 1
    ncpc = 1 if single_core else nchunks // NCORES

    @functools.partial(
        pl.kernel,
        out_type=jax.ShapeDtypeStruct((nchunks, NPAD, dc), dtype),
        mesh=_MESH,
        scratch_types=[
            pltpu.VMEM((NBLK, EB), jnp.int32),
            pltpu.VMEM((NBLK, EB), jnp.int32),
            pltpu.VMEM((EB, dc), dtype),
            pltpu.VMEM((EB, dc), dtype),
            pltpu.VMEM_SHARED((NPAD, dc), dtype),
            pltpu.SemaphoreType.DMA,
            pltpu.SemaphoreType.DMA,
        ],
        compiler_params=_SC_PARAMS,
    )
    def agg(y_hbm, src_hbm, dst_hbm, out_hbm, src_v, dst_v, buf0, buf1,
            acc_sh, sem0, sem1):
        c = lax.axis_index("c")
        s = lax.axis_index("s")
        rb = s * ROWS_PT

        def body():
            pltpu.sync_copy(src_hbm.at[s], src_v)
            pltpu.sync_copy(dst_hbm.at[s], dst_v)
            for p in range(ncpc):
                chunk = 0 if single_core else p * NCORES + c
                pltpu.sync_copy(y_hbm.at[chunk].at[pl.ds(rb, ROWS_PT)],
                                acc_sh.at[pl.ds(rb, ROWS_PT)])
                pltpu.async_copy(y_hbm.at[chunk].at[src_v.at[0]], buf0, sem0)
                plsc.subcore_barrier()

                def outer(t, carry):
                    jj = 2 * t
                    pltpu.async_copy(y_hbm.at[chunk].at[src_v.at[jj + 1]],
                                     buf1, sem1)
                    pltpu.make_async_copy(
                        y_hbm.at[chunk].at[src_v.at[jj]], buf0, sem0).wait()
                    pltpu.sync_copy(buf0, acc_sh.at[dst_v.at[jj]], add=True)

                    @pl.when(jj + 2 < NBLK)
                    def _():
                        pltpu.async_copy(y_hbm.at[chunk].at[src_v.at[jj + 2]],
                                         buf0, sem0)

                    pltpu.make_async_copy(
                        y_hbm.at[chunk].at[src_v.at[jj + 1]], buf1, sem1).wait()
                    pltpu.sync_copy(buf1, acc_sh.at[dst_v.at[jj + 1]], add=True)
                    return carry

                lax.fori_loop(0, NBLK // 2, outer, 0)
                plsc.subcore_barrier()
                pltpu.sync_copy(acc_sh.at[pl.ds(rb, ROWS_PT)],
                                out_hbm.at[chunk].at[pl.ds(rb, ROWS_PT)])
                plsc.subcore_barrier()

        if single_core:
            @pl.when(c == 0)
            def _():
                body()
        else:
            body()

    return agg


@functools.partial(
    pl.kernel,
    out_type=jax.ShapeDtypeStruct((NPAD, 16), jnp.float32),
    mesh=_MESH,
    scratch_types=[
        pltpu.VMEM((NBLK, EB), jnp.int32),
        pltpu.VMEM((EB, 16), jnp.float32),
        pltpu.VMEM_SHARED((NPAD, 16), jnp.float32),
    ],
    compiler_params=_SC_PARAMS,
)
def _deg_kernel(zeros_hbm, ones_hbm, dst_hbm, out_hbm, dst_v, ones_v, acc_sh):
    c = lax.axis_index("c")
    s = lax.axis_index("s")
    rb = s * ROWS_PT

    @pl.when(c == 0)
    def _():
        pltpu.sync_copy(dst_hbm.at[s], dst_v)
        pltpu.sync_copy(ones_hbm, ones_v)
        pltpu.sync_copy(zeros_hbm.at[pl.ds(rb, ROWS_PT)],
                        acc_sh.at[pl.ds(rb, ROWS_PT)])
        plsc.subcore_barrier()

        def blk(j, carry):
            pltpu.sync_copy(ones_v, acc_sh.at[dst_v.at[j]], add=True)
            return carry

        lax.fori_loop(0, NBLK, blk, 0)
        plsc.subcore_barrier()
        pltpu.sync_copy(acc_sh.at[pl.ds(rb, ROWS_PT)],
                        out_hbm.at[pl.ds(rb, ROWS_PT)])


def _prep_kernel(deg_ref, pos_ref, dinv_ref, y0_ref):
    i = pl.program_id(0)
    rows = i * BN + lax.broadcasted_iota(jnp.int32, (BN, 1), 0)
    dv = lax.rsqrt(deg_ref[:, :1] + 1.0)
    dv = jnp.where(rows < N, dv, 0.0)
    dinv_ref[...] = dv
    y0_ref[0] = pos_ref[...] * dv


_prep = pl.pallas_call(
    _prep_kernel,
    grid=(GRID_N,),
    in_specs=[
        pl.BlockSpec((BN, 16), lambda i: (i, 0)),
        pl.BlockSpec((BN, 16), lambda i: (i, 0)),
    ],
    out_specs=[
        pl.BlockSpec((BN, 1), lambda i: (i, 0)),
        pl.BlockSpec((1, BN, 16), lambda i: (0, i, 0)),
    ],
    out_shape=[
        jax.ShapeDtypeStruct((NPAD, 1), jnp.float32),
        jax.ShapeDtypeStruct((1, NPAD, 16), jnp.float32),
    ],
)


def _make_gemm(nc_in, dc_in, nc_out, dc_out,
               in_dtype=jnp.float32, out_dtype=jnp.float32):
    d_out = nc_out * dc_out

    def kern(acc_ref, dinv_ref, w_ref, b_ref, y_ref):
        d = dinv_ref[...]
        z = jnp.zeros((BN, d_out), jnp.float32)
        for ci in range(nc_in):
            x = acc_ref[ci].astype(jnp.float32) * d
            z = z + jnp.dot(x, w_ref[ci],
                            preferred_element_type=jnp.float32)
        z = z + b_ref[...]
        h = jnp.where(z >= 0, z, 0.01 * z)
        y = h * d
        for co in range(nc_out):
            y_ref[co] = y[:, co * dc_out:(co + 1) * dc_out].astype(out_dtype)

    return pl.pallas_call(
        kern,
        grid=(GRID_N,),
        in_specs=[
            pl.BlockSpec((nc_in, BN, dc_in), lambda i: (0, i, 0)),
            pl.BlockSpec((BN, 1), lambda i: (i, 0)),
            pl.BlockSpec((nc_in, dc_in, d_out), lambda i: (0, 0, 0)),
            pl.BlockSpec((1, d_out), lambda i: (0, 0)),
        ],
        out_specs=pl.BlockSpec((nc_out, BN, dc_out), lambda i: (0, i, 0)),
        out_shape=jax.ShapeDtypeStruct((nc_out, NPAD, dc_out), out_dtype),
    )


def _final_kernel(acc_ref, dinv_ref, w_ref, b_ref, fcw_ref, fcb_ref,
                  out_ref, r_sc):
    i = pl.program_id(0)
    d = dinv_ref[...]
    z = jnp.zeros((BN, 1024), jnp.float32)
    for ci in range(4):
        z = z + jnp.dot(acc_ref[ci].astype(jnp.float32) * d, w_ref[ci],
                        preferred_element_type=jnp.float32)
    z = z + b_ref[...]
    h = jnp.where(z >= 0, z, 0.01 * z)
    rows = i * BN + lax.broadcasted_iota(jnp.int32, (BN, 1), 0)
    hm = jnp.where(rows < N, h, -jnp.inf)
    m = jnp.max(hm, axis=0, keepdims=True)

    @pl.when(i == 0)
    def _():
        r_sc[...] = m

    @pl.when(i > 0)
    def _():
        r_sc[...] = jnp.maximum(r_sc[...], m)

    @pl.when(i == GRID_N - 1)
    def _():
        r = r_sc[...]
        out_ref[...] = lax.dot_general(
            r, fcw_ref[...], (((1,), (1,)), ((), ())),
            preferred_element_type=jnp.float32) + fcb_ref[...]


_final = pl.pallas_call(
    _final_kernel,
    grid=(GRID_N,),
    in_specs=[
        pl.BlockSpec((4, BN, 128), lambda i: (0, i, 0)),
        pl.BlockSpec((BN, 1), lambda i: (i, 0)),
        pl.BlockSpec((4, 128, 1024), lambda i: (0, 0, 0)),
        pl.BlockSpec((1, 1024), lambda i: (0, 0)),
        pl.BlockSpec((1024, 1024), lambda i: (0, 0)),
        pl.BlockSpec((1, 1024), lambda i: (0, 0)),
    ],
    out_specs=pl.BlockSpec((1, 1024), lambda i: (0, 0)),
    out_shape=jax.ShapeDtypeStruct((1, 1024), jnp.float32),
    scratch_shapes=[pltpu.VMEM((1, 1024), jnp.float32)],
    compiler_params=pltpu.CompilerParams(
        dimension_semantics=("arbitrary",)),
)

_agg1 = _make_agg(1, 16)
_agg2 = _make_agg(2, 32)
_agg3 = _make_agg(2, 64)
_agg4 = _make_agg(2, 128, jnp.bfloat16)
_agg5 = _make_agg(4, 128, jnp.bfloat16)
_gemm1 = _make_gemm(1, 16, 2, 32)
_gemm2 = _make_gemm(2, 32, 2, 64)
_gemm3 = _make_gemm(2, 64, 2, 128, out_dtype=jnp.bfloat16)
_gemm4 = _make_gemm(2, 128, 4, 128,
                    in_dtype=jnp.bfloat16, out_dtype=jnp.bfloat16)


def kernel(pos, edge_index, W1, b1, W2, b2, W3, b3, W4, b4, W5, b5, fcW, fcb):
    src = edge_index[0]
    dst = edge_index[1]
    pad = N + jnp.arange(EPAD - E, dtype=jnp.int32) % (NPAD - N)
    srcp = jnp.concatenate([src, pad]).reshape(NS, NBLK, EB)
    dstp = jnp.concatenate([dst, pad]).reshape(NS, NBLK, EB)
    pos_pad = jnp.pad(pos, ((0, NPAD - N), (0, 13)))
    zeros16 = jnp.zeros((NPAD, 16), jnp.float32)
    ones_rows = jnp.ones((EB, 16), jnp.float32)

    deg = _deg_kernel(zeros16, ones_rows, dstp)
    dinv, y0 = _prep(deg, pos_pad)

    a1 = _agg1(y0, srcp, dstp)
    y1 = _gemm1(a1, dinv, jnp.pad(W1, ((0, 13), (0, 0)))[None], b1[None])
    a2 = _agg2(y1, srcp, dstp)
    y2 = _gemm2(a2, dinv, W2.reshape(2, 32, 128), b2[None])
    a3 = _agg3(y2, srcp, dstp)
    y3 = _gemm3(a3, dinv, W3.reshape(2, 64, 256), b3[None])
    a4 = _agg4(y3, srcp, dstp)
    y4 = _gemm4(a4, dinv, W4.reshape(2, 128, 512), b4[None])
    a5 = _agg5(y4, srcp, dstp)
    out = _final(a5, dinv, W5.reshape(4, 128, 1024), b5[None], fcW, fcb[None])
    return out.reshape(1024)


# speedup vs baseline: 1.1574x; 1.1574x over previous
import functools
import jax
import jax.numpy as jnp
from jax import lax
from jax.experimental import pallas as pl
from jax.experimental.pallas import tpu as pltpu
from jax.experimental.pallas import tpu_sc as plsc

N = 10000
NPAD = 10240
E = 160000
NS = 16
NCORES = 2
EB = 64
NBLK = 160
EPT = NBLK * EB
EPAD = EPT * NS
ROWS_PT = NPAD // NS
BN = 1024
GRID_N = NPAD // BN

_SC_PARAMS = pltpu.CompilerParams(use_tc_tiling_on_sc=False)
_MESH = plsc.VectorSubcoreMesh(core_axis_name="c", subcore_axis_name="s")


def _make_agg(nchunks, dc, dtype=jnp.float32):
    edge_split = nchunks == 1
    ncpc = 1 if edge_split else nchunks // NCORES

    nout = NCORES if edge_split else nchunks

    @functools.partial(
        pl.kernel,
        out_type=jax.ShapeDtypeStruct((nout, NPAD, dc), dtype),
        mesh=_MESH,
        scratch_types=[
            pltpu.VMEM((NBLK, EB), jnp.int32),
            pltpu.VMEM((NBLK, EB), jnp.int32),
            pltpu.VMEM((EB, dc), dtype),
            pltpu.VMEM((EB, dc), dtype),
            pltpu.VMEM_SHARED((NPAD, dc), dtype),
            pltpu.SemaphoreType.DMA,
            pltpu.SemaphoreType.DMA,
        ],
        compiler_params=_SC_PARAMS,
    )
    def agg(y_hbm, zeros_hbm, src_hbm, dst_hbm, out_hbm, src_v, dst_v,
            buf0, buf1, acc_sh, sem0, sem1):
        c = lax.axis_index("c")
        s = lax.axis_index("s")
        rb = s * ROWS_PT
        pltpu.sync_copy(src_hbm.at[s], src_v)
        pltpu.sync_copy(dst_hbm.at[s], dst_v)
        for p in range(ncpc):
            if edge_split:
                chunk = 0
                oidx = c
                lo = c * (NBLK // 2)
            else:
                chunk = p * NCORES + c
                oidx = chunk
                lo = 0
            hi = lo + (NBLK if not edge_split else NBLK // 2)

            if edge_split:
                @pl.when(c == 0)
                def _():
                    pltpu.sync_copy(y_hbm.at[0].at[pl.ds(rb, ROWS_PT)],
                                    acc_sh.at[pl.ds(rb, ROWS_PT)])

                @pl.when(c == 1)
                def _():
                    pltpu.sync_copy(zeros_hbm.at[pl.ds(rb, ROWS_PT)],
                                    acc_sh.at[pl.ds(rb, ROWS_PT)])
            else:
                pltpu.sync_copy(y_hbm.at[chunk].at[pl.ds(rb, ROWS_PT)],
                                acc_sh.at[pl.ds(rb, ROWS_PT)])
            pltpu.async_copy(y_hbm.at[chunk].at[src_v.at[lo]], buf0, sem0)
            plsc.subcore_barrier()

            def outer(t, carry):
                jj = 2 * t
                pltpu.async_copy(y_hbm.at[chunk].at[src_v.at[jj + 1]],
                                 buf1, sem1)
                pltpu.make_async_copy(
                    y_hbm.at[chunk].at[src_v.at[jj]], buf0, sem0).wait()
                pltpu.sync_copy(buf0, acc_sh.at[dst_v.at[jj]], add=True)

                @pl.when(jj + 2 < hi)
                def _():
                    pltpu.async_copy(y_hbm.at[chunk].at[src_v.at[jj + 2]],
                                     buf0, sem0)

                pltpu.make_async_copy(
                    y_hbm.at[chunk].at[src_v.at[jj + 1]], buf1, sem1).wait()
                pltpu.sync_copy(buf1, acc_sh.at[dst_v.at[jj + 1]], add=True)
                return carry

            lax.fori_loop(lo // 2, hi // 2, outer, 0)
            plsc.subcore_barrier()
            pltpu.sync_copy(acc_sh.at[pl.ds(rb, ROWS_PT)],
                            out_hbm.at[oidx].at[pl.ds(rb, ROWS_PT)])
            plsc.subcore_barrier()

    return agg


@functools.partial(
    pl.kernel,
    out_type=jax.ShapeDtypeStruct((NPAD, 16), jnp.float32),
    mesh=_MESH,
    scratch_types=[
        pltpu.VMEM((NBLK, EB), jnp.int32),
        pltpu.VMEM((EB, 16), jnp.float32),
        pltpu.VMEM_SHARED((NPAD, 16), jnp.float32),
    ],
    compiler_params=_SC_PARAMS,
)
def _deg_kernel(zeros_hbm, ones_hbm, dst_hbm, out_hbm, dst_v, ones_v, acc_sh):
    c = lax.axis_index("c")
    s = lax.axis_index("s")
    rb = s * ROWS_PT

    @pl.when(c == 0)
    def _():
        pltpu.sync_copy(dst_hbm.at[s], dst_v)
        pltpu.sync_copy(ones_hbm, ones_v)
        pltpu.sync_copy(zeros_hbm.at[pl.ds(rb, ROWS_PT)],
                        acc_sh.at[pl.ds(rb, ROWS_PT)])
        plsc.subcore_barrier()

        def blk(j, carry):
            pltpu.sync_copy(ones_v, acc_sh.at[dst_v.at[j]], add=True)
            return carry

        lax.fori_loop(0, NBLK, blk, 0)
        plsc.subcore_barrier()
        pltpu.sync_copy(acc_sh.at[pl.ds(rb, ROWS_PT)],
                        out_hbm.at[pl.ds(rb, ROWS_PT)])


def _prep_kernel(deg_ref, pos_ref, dinv_ref, y0_ref):
    i = pl.program_id(0)
    rows = i * BN + lax.broadcasted_iota(jnp.int32, (BN, 1), 0)
    dv = lax.rsqrt(deg_ref[:, :1] + 1.0)
    dv = jnp.where(rows < N, dv, 0.0)
    dinv_ref[...] = dv
    y0_ref[0] = pos_ref[...] * dv


_prep = pl.pallas_call(
    _prep_kernel,
    grid=(GRID_N,),
    in_specs=[
        pl.BlockSpec((BN, 16), lambda i: (i, 0)),
        pl.BlockSpec((BN, 16), lambda i: (i, 0)),
    ],
    out_specs=[
        pl.BlockSpec((BN, 1), lambda i: (i, 0)),
        pl.BlockSpec((1, BN, 16), lambda i: (0, i, 0)),
    ],
    out_shape=[
        jax.ShapeDtypeStruct((NPAD, 1), jnp.float32),
        jax.ShapeDtypeStruct((1, NPAD, 16), jnp.float32),
    ],
)


def _make_gemm(nc_in, dc_in, nc_out, dc_out,
               in_dtype=jnp.float32, out_dtype=jnp.float32, sum_in=False):
    d_out = nc_out * dc_out

    def kern(acc_ref, dinv_ref, w_ref, b_ref, y_ref):
        d = dinv_ref[...]
        if sum_in:
            x = (acc_ref[0].astype(jnp.float32)
                 + acc_ref[1].astype(jnp.float32)) * d
            z = jnp.dot(x, w_ref[0], preferred_element_type=jnp.float32)
        else:
            z = jnp.zeros((BN, d_out), jnp.float32)
            for ci in range(nc_in):
                x = acc_ref[ci].astype(jnp.float32) * d
                z = z + jnp.dot(x, w_ref[ci],
                                preferred_element_type=jnp.float32)
        z = z + b_ref[...]
        h = jnp.where(z >= 0, z, 0.01 * z)
        y = h * d
        for co in range(nc_out):
            y_ref[co] = y[:, co * dc_out:(co + 1) * dc_out].astype(out_dtype)

    return pl.pallas_call(
        kern,
        grid=(GRID_N,),
        in_specs=[
            pl.BlockSpec((2 if sum_in else nc_in, BN, dc_in),
                         lambda i: (0, i, 0)),
            pl.BlockSpec((BN, 1), lambda i: (i, 0)),
            pl.BlockSpec((1 if sum_in else nc_in, dc_in, d_out),
                         lambda i: (0, 0, 0)),
            pl.BlockSpec((1, d_out), lambda i: (0, 0)),
        ],
        out_specs=pl.BlockSpec((nc_out, BN, dc_out), lambda i: (0, i, 0)),
        out_shape=jax.ShapeDtypeStruct((nc_out, NPAD, dc_out), out_dtype),
    )


def _final_kernel(acc_ref, dinv_ref, w_ref, b_ref, fcw_ref, fcb_ref,
                  out_ref, r_sc):
    i = pl.program_id(0)
    d = dinv_ref[...]
    z = jnp.zeros((BN, 1024), jnp.float32)
    for ci in range(4):
        z = z + jnp.dot(acc_ref[ci].astype(jnp.float32) * d, w_ref[ci],
                        preferred_element_type=jnp.float32)
    z = z + b_ref[...]
    h = jnp.where(z >= 0, z, 0.01 * z)
    rows = i * BN + lax.broadcasted_iota(jnp.int32, (BN, 1), 0)
    hm = jnp.where(rows < N, h, -jnp.inf)
    m = jnp.max(hm, axis=0, keepdims=True)

    @pl.when(i == 0)
    def _():
        r_sc[...] = m

    @pl.when(i > 0)
    def _():
        r_sc[...] = jnp.maximum(r_sc[...], m)

    @pl.when(i == GRID_N - 1)
    def _():
        r = r_sc[...]
        out_ref[...] = lax.dot_general(
            r, fcw_ref[...], (((1,), (1,)), ((), ())),
            preferred_element_type=jnp.float32) + fcb_ref[...]


_final = pl.pallas_call(
    _final_kernel,
    grid=(GRID_N,),
    in_specs=[
        pl.BlockSpec((4, BN, 128), lambda i: (0, i, 0)),
        pl.BlockSpec((BN, 1), lambda i: (i, 0)),
        pl.BlockSpec((4, 128, 1024), lambda i: (0, 0, 0)),
        pl.BlockSpec((1, 1024), lambda i: (0, 0)),
        pl.BlockSpec((1024, 1024), lambda i: (0, 0)),
        pl.BlockSpec((1, 1024), lambda i: (0, 0)),
    ],
    out_specs=pl.BlockSpec((1, 1024), lambda i: (0, 0)),
    out_shape=jax.ShapeDtypeStruct((1, 1024), jnp.float32),
    scratch_shapes=[pltpu.VMEM((1, 1024), jnp.float32)],
    compiler_params=pltpu.CompilerParams(
        dimension_semantics=("arbitrary",)),
)

_agg1 = _make_agg(1, 16)
_agg2 = _make_agg(2, 32)
_agg3 = _make_agg(2, 64)
_agg4 = _make_agg(2, 128)
_agg5 = _make_agg(4, 128)
_gemm1 = _make_gemm(1, 16, 2, 32, sum_in=True)
_gemm2 = _make_gemm(2, 32, 2, 64)
_gemm3 = _make_gemm(2, 64, 2, 128)
_gemm4 = _make_gemm(2, 128, 4, 128)


def kernel(pos, edge_index, W1, b1, W2, b2, W3, b3, W4, b4, W5, b5, fcW, fcb):
    src = edge_index[0]
    dst = edge_index[1]
    pad = N + jnp.arange(EPAD - E, dtype=jnp.int32) % (NPAD - N)
    srcp = jnp.concatenate([src, pad]).reshape(NS, NBLK, EB)
    dstp = jnp.concatenate([dst, pad]).reshape(NS, NBLK, EB)
    pos_pad = jnp.pad(pos, ((0, NPAD - N), (0, 13)))
    zeros16 = jnp.zeros((NPAD, 16), jnp.float32)
    ones_rows = jnp.ones((EB, 16), jnp.float32)

    deg = _deg_kernel(zeros16, ones_rows, dstp)
    dinv, y0 = _prep(deg, pos_pad)

    a1 = _agg1(y0, zeros16, srcp, dstp)
    y1 = _gemm1(a1, dinv, jnp.pad(W1, ((0, 13), (0, 0)))[None], b1[None])
    a2 = _agg2(y1, zeros16, srcp, dstp)
    y2 = _gemm2(a2, dinv, W2.reshape(2, 32, 128), b2[None])
    a3 = _agg3(y2, zeros16, srcp, dstp)
    y3 = _gemm3(a3, dinv, W3.reshape(2, 64, 256), b3[None])
    a4 = _agg4(y3, zeros16, srcp, dstp)
    y4 = _gemm4(a4, dinv, W4.reshape(2, 128, 512), b4[None])
    a5 = _agg5(y4, zeros16, srcp, dstp)
    out = _final(a5, dinv, W5.reshape(4, 128, 1024), b5[None], fcW, fcb[None])
    return out.reshape(1024)


# speedup vs baseline: 1.2491x; 1.0792x over previous
import functools
import jax
import jax.numpy as jnp
from jax import lax
from jax.experimental import pallas as pl
from jax.experimental.pallas import tpu as pltpu
from jax.experimental.pallas import tpu_sc as plsc

N = 10000
NPAD = 10240
E = 160000
NS = 16
NCORES = 2
EB = 80
NBLK = 128
EPT = NBLK * EB
EPAD = EPT * NS
ROWS_PT = NPAD // NS
BN = 1024
GRID_N = NPAD // BN

_SC_PARAMS = pltpu.CompilerParams(use_tc_tiling_on_sc=False)
_MESH = plsc.VectorSubcoreMesh(core_axis_name="c", subcore_axis_name="s")


def _make_agg(nchunks, dc, dtype=jnp.float32):
    edge_split = nchunks == 1
    ncpc = 1 if edge_split else nchunks // NCORES

    nout = NCORES if edge_split else nchunks

    @functools.partial(
        pl.kernel,
        out_type=jax.ShapeDtypeStruct((nout, NPAD, dc), dtype),
        mesh=_MESH,
        scratch_types=[
            pltpu.VMEM((NBLK, EB), jnp.int32),
            pltpu.VMEM((NBLK, EB), jnp.int32),
            pltpu.VMEM((EB, dc), dtype),
            pltpu.VMEM((EB, dc), dtype),
            pltpu.VMEM_SHARED((NPAD, dc), dtype),
            pltpu.SemaphoreType.DMA,
            pltpu.SemaphoreType.DMA,
        ],
        compiler_params=_SC_PARAMS,
    )
    def agg(y_hbm, zeros_hbm, src_hbm, dst_hbm, out_hbm, src_v, dst_v,
            buf0, buf1, acc_sh, sem0, sem1):
        c = lax.axis_index("c")
        s = lax.axis_index("s")
        rb = s * ROWS_PT
        pltpu.sync_copy(src_hbm.at[s], src_v)
        pltpu.sync_copy(dst_hbm.at[s], dst_v)
        for p in range(ncpc):
            if edge_split:
                chunk = 0
                oidx = c
                lo = c * (NBLK // 2)
            else:
                chunk = p * NCORES + c
                oidx = chunk
                lo = 0
            hi = lo + (NBLK if not edge_split else NBLK // 2)

            if edge_split:
                @pl.when(c == 0)
                def _():
                    pltpu.sync_copy(y_hbm.at[0].at[pl.ds(rb, ROWS_PT)],
                                    acc_sh.at[pl.ds(rb, ROWS_PT)])

                @pl.when(c == 1)
                def _():
                    pltpu.sync_copy(zeros_hbm.at[pl.ds(rb, ROWS_PT)],
                                    acc_sh.at[pl.ds(rb, ROWS_PT)])
            else:
                pltpu.sync_copy(y_hbm.at[chunk].at[pl.ds(rb, ROWS_PT)],
                                acc_sh.at[pl.ds(rb, ROWS_PT)])
            bufs = (buf0, buf1)
            sems = (sem0, sem1)
            for k in range(2):
                pltpu.async_copy(y_hbm.at[chunk].at[src_v.at[lo + k]],
                                 bufs[k], sems[k])
            plsc.subcore_barrier()

            def outer(t, carry):
                base = 2 * t
                for k in range(2):
                    m = base + k
                    pltpu.make_async_copy(
                        y_hbm.at[chunk].at[src_v.at[m]],
                        bufs[k], sems[k]).wait()
                    pltpu.sync_copy(bufs[k], acc_sh.at[dst_v.at[m]],
                                    add=True)

                    @pl.when(m + 2 < hi)
                    def _():
                        pltpu.async_copy(y_hbm.at[chunk].at[src_v.at[m + 2]],
                                         bufs[k], sems[k])
                return carry

            lax.fori_loop(lo // 2, hi // 2, outer, 0)
            plsc.subcore_barrier()
            pltpu.sync_copy(acc_sh.at[pl.ds(rb, ROWS_PT)],
                            out_hbm.at[oidx].at[pl.ds(rb, ROWS_PT)])
            plsc.subcore_barrier()

    return agg


@functools.partial(
    pl.kernel,
    out_type=jax.ShapeDtypeStruct((NPAD, 16), jnp.float32),
    mesh=_MESH,
    scratch_types=[
        pltpu.VMEM((NBLK, EB), jnp.int32),
        pltpu.VMEM((EB, 16), jnp.float32),
        pltpu.VMEM_SHARED((NPAD, 16), jnp.float32),
    ],
    compiler_params=_SC_PARAMS,
)
def _deg_kernel(zeros_hbm, ones_hbm, dst_hbm, out_hbm, dst_v, ones_v, acc_sh):
    c = lax.axis_index("c")
    s = lax.axis_index("s")
    rb = s * ROWS_PT

    @pl.when(c == 0)
    def _():
        pltpu.sync_copy(dst_hbm.at[s], dst_v)
        pltpu.sync_copy(ones_hbm, ones_v)
        pltpu.sync_copy(zeros_hbm.at[pl.ds(rb, ROWS_PT)],
                        acc_sh.at[pl.ds(rb, ROWS_PT)])
        plsc.subcore_barrier()

        def blk(j, carry):
            pltpu.sync_copy(ones_v, acc_sh.at[dst_v.at[j]], add=True)
            return carry

        lax.fori_loop(0, NBLK, blk, 0)
        plsc.subcore_barrier()
        pltpu.sync_copy(acc_sh.at[pl.ds(rb, ROWS_PT)],
                        out_hbm.at[pl.ds(rb, ROWS_PT)])


def _prep_kernel(deg_ref, pos_ref, dinv_ref, y0_ref):
    i = pl.program_id(0)
    rows = i * BN + lax.broadcasted_iota(jnp.int32, (BN, 1), 0)
    dv = lax.rsqrt(deg_ref[:, :1] + 1.0)
    dv = jnp.where(rows < N, dv, 0.0)
    dinv_ref[...] = dv
    y0_ref[0] = pos_ref[...] * dv


_prep = pl.pallas_call(
    _prep_kernel,
    grid=(GRID_N,),
    in_specs=[
        pl.BlockSpec((BN, 16), lambda i: (i, 0)),
        pl.BlockSpec((BN, 16), lambda i: (i, 0)),
    ],
    out_specs=[
        pl.BlockSpec((BN, 1), lambda i: (i, 0)),
        pl.BlockSpec((1, BN, 16), lambda i: (0, i, 0)),
    ],
    out_shape=[
        jax.ShapeDtypeStruct((NPAD, 1), jnp.float32),
        jax.ShapeDtypeStruct((1, NPAD, 16), jnp.float32),
    ],
)


def _make_gemm(nc_in, dc_in, nc_out, dc_out,
               in_dtype=jnp.float32, out_dtype=jnp.float32, sum_in=False):
    d_out = nc_out * dc_out

    def kern(acc_ref, dinv_ref, w_ref, b_ref, y_ref):
        d = dinv_ref[...]
        if sum_in:
            x = (acc_ref[0].astype(jnp.float32)
                 + acc_ref[1].astype(jnp.float32)) * d
            z = jnp.dot(x, w_ref[0], preferred_element_type=jnp.float32)
        else:
            z = jnp.zeros((BN, d_out), jnp.float32)
            for ci in range(nc_in):
                x = acc_ref[ci].astype(jnp.float32) * d
                z = z + jnp.dot(x, w_ref[ci],
                                preferred_element_type=jnp.float32)
        z = z + b_ref[...]
        h = jnp.where(z >= 0, z, 0.01 * z)
        y = h * d
        for co in range(nc_out):
            y_ref[co] = y[:, co * dc_out:(co + 1) * dc_out].astype(out_dtype)

    return pl.pallas_call(
        kern,
        grid=(GRID_N,),
        in_specs=[
            pl.BlockSpec((2 if sum_in else nc_in, BN, dc_in),
                         lambda i: (0, i, 0)),
            pl.BlockSpec((BN, 1), lambda i: (i, 0)),
            pl.BlockSpec((1 if sum_in else nc_in, dc_in, d_out),
                         lambda i: (0, 0, 0)),
            pl.BlockSpec((1, d_out), lambda i: (0, 0)),
        ],
        out_specs=pl.BlockSpec((nc_out, BN, dc_out), lambda i: (0, i, 0)),
        out_shape=jax.ShapeDtypeStruct((nc_out, NPAD, dc_out), out_dtype),
    )


def _final_kernel(acc_ref, dinv_ref, w_ref, b_ref, fcw_ref, fcb_ref,
                  out_ref, r_sc):
    i = pl.program_id(0)
    d = dinv_ref[...]
    z = jnp.zeros((BN, 1024), jnp.float32)
    for ci in range(4):
        z = z + jnp.dot(acc_ref[ci].astype(jnp.float32) * d, w_ref[ci],
                        preferred_element_type=jnp.float32)
    z = z + b_ref[...]
    h = jnp.where(z >= 0, z, 0.01 * z)
    rows = i * BN + lax.broadcasted_iota(jnp.int32, (BN, 1), 0)
    hm = jnp.where(rows < N, h, -jnp.inf)
    m = jnp.max(hm, axis=0, keepdims=True)

    @pl.when(i == 0)
    def _():
        r_sc[...] = m

    @pl.when(i > 0)
    def _():
        r_sc[...] = jnp.maximum(r_sc[...], m)

    @pl.when(i == GRID_N - 1)
    def _():
        r = r_sc[...]
        out_ref[...] = lax.dot_general(
            r, fcw_ref[...], (((1,), (1,)), ((), ())),
            preferred_element_type=jnp.float32) + fcb_ref[...]


_final = pl.pallas_call(
    _final_kernel,
    grid=(GRID_N,),
    in_specs=[
        pl.BlockSpec((4, BN, 128), lambda i: (0, i, 0)),
        pl.BlockSpec((BN, 1), lambda i: (i, 0)),
        pl.BlockSpec((4, 128, 1024), lambda i: (0, 0, 0)),
        pl.BlockSpec((1, 1024), lambda i: (0, 0)),
        pl.BlockSpec((1024, 1024), lambda i: (0, 0)),
        pl.BlockSpec((1, 1024), lambda i: (0, 0)),
    ],
    out_specs=pl.BlockSpec((1, 1024), lambda i: (0, 0)),
    out_shape=jax.ShapeDtypeStruct((1, 1024), jnp.float32),
    scratch_shapes=[pltpu.VMEM((1, 1024), jnp.float32)],
    compiler_params=pltpu.CompilerParams(
        dimension_semantics=("arbitrary",)),
)

_agg1 = _make_agg(1, 16)
_agg2 = _make_agg(2, 32)
_agg3 = _make_agg(2, 64)
_agg4 = _make_agg(2, 128)
_agg5 = _make_agg(4, 128)
_gemm1 = _make_gemm(1, 16, 2, 32, sum_in=True)
_gemm2 = _make_gemm(2, 32, 2, 64)
_gemm3 = _make_gemm(2, 64, 2, 128)
_gemm4 = _make_gemm(2, 128, 4, 128)


def kernel(pos, edge_index, W1, b1, W2, b2, W3, b3, W4, b4, W5, b5, fcW, fcb):
    src = edge_index[0]
    dst = edge_index[1]
    pad = N + jnp.arange(EPAD - E, dtype=jnp.int32) % (NPAD - N)
    srcp = jnp.concatenate([src, pad]).reshape(NS, NBLK, EB)
    dstp = jnp.concatenate([dst, pad]).reshape(NS, NBLK, EB)
    pos_pad = jnp.pad(pos, ((0, NPAD - N), (0, 13)))
    zeros16 = jnp.zeros((NPAD, 16), jnp.float32)
    ones_rows = jnp.ones((EB, 16), jnp.float32)

    deg = _deg_kernel(zeros16, ones_rows, dstp)
    dinv, y0 = _prep(deg, pos_pad)

    a1 = _agg1(y0, zeros16, srcp, dstp)
    y1 = _gemm1(a1, dinv, jnp.pad(W1, ((0, 13), (0, 0)))[None], b1[None])
    a2 = _agg2(y1, zeros16, srcp, dstp)
    y2 = _gemm2(a2, dinv, W2.reshape(2, 32, 128), b2[None])
    a3 = _agg3(y2, zeros16, srcp, dstp)
    y3 = _gemm3(a3, dinv, W3.reshape(2, 64, 256), b3[None])
    a4 = _agg4(y3, zeros16, srcp, dstp)
    y4 = _gemm4(a4, dinv, W4.reshape(2, 128, 512), b4[None])
    a5 = _agg5(y4, zeros16, srcp, dstp)
    out = _final(a5, dinv, W5.reshape(4, 128, 1024), b5[None], fcW, fcb[None])
    return out.reshape(1024)


# speedup vs baseline: 1.3185x; 1.0556x over previous
import functools
import jax
import jax.numpy as jnp
from jax import lax
from jax.experimental import pallas as pl
from jax.experimental.pallas import tpu as pltpu
from jax.experimental.pallas import tpu_sc as plsc

N = 10000
NPAD = 10240
E = 160000
NS = 16
NCORES = 2
EPT = 10240
EB = 128
NBLK = EPT // EB
EB_W = 80
NBLK_W = EPT // EB_W
EPAD = EPT * NS
ROWS_PT = NPAD // NS
BN = 1024
GRID_N = NPAD // BN

_SC_PARAMS = pltpu.CompilerParams(use_tc_tiling_on_sc=False)
_MESH = plsc.VectorSubcoreMesh(core_axis_name="c", subcore_axis_name="s")


def _make_agg(nchunks, dc, dtype=jnp.float32):
    edge_split = nchunks == 1
    ncpc = 1 if edge_split else nchunks // NCORES

    nout = NCORES if edge_split else nchunks
    eb = EB_W if dc == 128 else EB
    nblk = EPT // eb

    @functools.partial(
        pl.kernel,
        out_type=jax.ShapeDtypeStruct((nout, NPAD, dc), dtype),
        mesh=_MESH,
        scratch_types=[
            pltpu.VMEM((nblk, eb), jnp.int32),
            pltpu.VMEM((nblk, eb), jnp.int32),
            pltpu.VMEM((eb, dc), dtype),
            pltpu.VMEM((eb, dc), dtype),
            pltpu.VMEM_SHARED((NPAD, dc), dtype),
            pltpu.SemaphoreType.DMA,
            pltpu.SemaphoreType.DMA,
        ],
        compiler_params=_SC_PARAMS,
    )
    def agg(y_hbm, zeros_hbm, src_hbm, dst_hbm, out_hbm, src_v, dst_v,
            buf0, buf1, acc_sh, sem0, sem1):
        c = lax.axis_index("c")
        s = lax.axis_index("s")
        rb = s * ROWS_PT
        pltpu.sync_copy(src_hbm.at[s], src_v)
        pltpu.sync_copy(dst_hbm.at[s], dst_v)
        for p in range(ncpc):
            if edge_split:
                chunk = 0
                oidx = c
                lo = c * (nblk // 2)
            else:
                chunk = p * NCORES + c
                oidx = chunk
                lo = 0
            hi = lo + (nblk if not edge_split else nblk // 2)

            if edge_split:
                @pl.when(c == 0)
                def _():
                    pltpu.sync_copy(y_hbm.at[0].at[pl.ds(rb, ROWS_PT)],
                                    acc_sh.at[pl.ds(rb, ROWS_PT)])

                @pl.when(c == 1)
                def _():
                    pltpu.sync_copy(zeros_hbm.at[pl.ds(rb, ROWS_PT)],
                                    acc_sh.at[pl.ds(rb, ROWS_PT)])
            else:
                pltpu.sync_copy(y_hbm.at[chunk].at[pl.ds(rb, ROWS_PT)],
                                acc_sh.at[pl.ds(rb, ROWS_PT)])
            bufs = (buf0, buf1)
            sems = (sem0, sem1)
            for k in range(2):
                pltpu.async_copy(y_hbm.at[chunk].at[src_v.at[lo + k]],
                                 bufs[k], sems[k])
            plsc.subcore_barrier()

            def outer(t, carry):
                base = 2 * t
                for k in range(2):
                    m = base + k
                    pltpu.make_async_copy(
                        y_hbm.at[chunk].at[src_v.at[m]],
                        bufs[k], sems[k]).wait()
                    pltpu.sync_copy(bufs[k], acc_sh.at[dst_v.at[m]],
                                    add=True)

                    @pl.when(m + 2 < hi)
                    def _():
                        pltpu.async_copy(y_hbm.at[chunk].at[src_v.at[m + 2]],
                                         bufs[k], sems[k])
                return carry

            lax.fori_loop(lo // 2, hi // 2, outer, 0)
            plsc.subcore_barrier()
            pltpu.sync_copy(acc_sh.at[pl.ds(rb, ROWS_PT)],
                            out_hbm.at[oidx].at[pl.ds(rb, ROWS_PT)])
            plsc.subcore_barrier()

    return agg


@functools.partial(
    pl.kernel,
    out_type=jax.ShapeDtypeStruct((NPAD, 16), jnp.float32),
    mesh=_MESH,
    scratch_types=[
        pltpu.VMEM((NBLK, EB), jnp.int32),
        pltpu.VMEM((EB, 16), jnp.float32),
        pltpu.VMEM_SHARED((NPAD, 16), jnp.float32),
    ],
    compiler_params=_SC_PARAMS,
)
def _deg_kernel(zeros_hbm, ones_hbm, dst_hbm, out_hbm, dst_v, ones_v, acc_sh):
    c = lax.axis_index("c")
    s = lax.axis_index("s")
    rb = s * ROWS_PT

    @pl.when(c == 0)
    def _():
        pltpu.sync_copy(dst_hbm.at[s], dst_v)
        pltpu.sync_copy(ones_hbm, ones_v)
        pltpu.sync_copy(zeros_hbm.at[pl.ds(rb, ROWS_PT)],
                        acc_sh.at[pl.ds(rb, ROWS_PT)])
        plsc.subcore_barrier()

        def blk(j, carry):
            pltpu.sync_copy(ones_v, acc_sh.at[dst_v.at[j]], add=True)
            return carry

        lax.fori_loop(0, NBLK, blk, 0)
        plsc.subcore_barrier()
        pltpu.sync_copy(acc_sh.at[pl.ds(rb, ROWS_PT)],
                        out_hbm.at[pl.ds(rb, ROWS_PT)])


def _prep_kernel(deg_ref, pos_ref, dinv_ref, y0_ref):
    i = pl.program_id(0)
    rows = i * BN + lax.broadcasted_iota(jnp.int32, (BN, 1), 0)
    dv = lax.rsqrt(deg_ref[:, :1] + 1.0)
    dv = jnp.where(rows < N, dv, 0.0)
    dinv_ref[...] = dv
    y0_ref[0] = pos_ref[...] * dv


_prep = pl.pallas_call(
    _prep_kernel,
    grid=(GRID_N,),
    in_specs=[
        pl.BlockSpec((BN, 16), lambda i: (i, 0)),
        pl.BlockSpec((BN, 16), lambda i: (i, 0)),
    ],
    out_specs=[
        pl.BlockSpec((BN, 1), lambda i: (i, 0)),
        pl.BlockSpec((1, BN, 16), lambda i: (0, i, 0)),
    ],
    out_shape=[
        jax.ShapeDtypeStruct((NPAD, 1), jnp.float32),
        jax.ShapeDtypeStruct((1, NPAD, 16), jnp.float32),
    ],
)


def _make_gemm(nc_in, dc_in, nc_out, dc_out,
               in_dtype=jnp.float32, out_dtype=jnp.float32, sum_in=False):
    d_out = nc_out * dc_out

    def kern(acc_ref, dinv_ref, w_ref, b_ref, y_ref):
        d = dinv_ref[...]
        if sum_in:
            x = (acc_ref[0].astype(jnp.float32)
                 + acc_ref[1].astype(jnp.float32)) * d
            z = jnp.dot(x, w_ref[0], preferred_element_type=jnp.float32)
        else:
            z = jnp.zeros((BN, d_out), jnp.float32)
            for ci in range(nc_in):
                x = acc_ref[ci].astype(jnp.float32) * d
                z = z + jnp.dot(x, w_ref[ci],
                                preferred_element_type=jnp.float32)
        z = z + b_ref[...]
        h = jnp.where(z >= 0, z, 0.01 * z)
        y = h * d
        for co in range(nc_out):
            y_ref[co] = y[:, co * dc_out:(co + 1) * dc_out].astype(out_dtype)

    return pl.pallas_call(
        kern,
        grid=(GRID_N,),
        in_specs=[
            pl.BlockSpec((2 if sum_in else nc_in, BN, dc_in),
                         lambda i: (0, i, 0)),
            pl.BlockSpec((BN, 1), lambda i: (i, 0)),
            pl.BlockSpec((1 if sum_in else nc_in, dc_in, d_out),
                         lambda i: (0, 0, 0)),
            pl.BlockSpec((1, d_out), lambda i: (0, 0)),
        ],
        out_specs=pl.BlockSpec((nc_out, BN, dc_out), lambda i: (0, i, 0)),
        out_shape=jax.ShapeDtypeStruct((nc_out, NPAD, dc_out), out_dtype),
    )


def _final_kernel(acc_ref, dinv_ref, w_ref, b_ref, fcw_ref, fcb_ref,
                  out_ref, r_sc):
    i = pl.program_id(0)
    d = dinv_ref[...]
    z = jnp.zeros((BN, 1024), jnp.float32)
    for ci in range(4):
        z = z + jnp.dot(acc_ref[ci].astype(jnp.float32) * d, w_ref[ci],
                        preferred_element_type=jnp.float32)
    z = z + b_ref[...]
    h = jnp.where(z >= 0, z, 0.01 * z)
    rows = i * BN + lax.broadcasted_iota(jnp.int32, (BN, 1), 0)
    hm = jnp.where(rows < N, h, -jnp.inf)
    m = jnp.max(hm, axis=0, keepdims=True)

    @pl.when(i == 0)
    def _():
        r_sc[...] = m

    @pl.when(i > 0)
    def _():
        r_sc[...] = jnp.maximum(r_sc[...], m)

    @pl.when(i == GRID_N - 1)
    def _():
        r = r_sc[...]
        out_ref[...] = lax.dot_general(
            r, fcw_ref[...], (((1,), (1,)), ((), ())),
            preferred_element_type=jnp.float32) + fcb_ref[...]


_final = pl.pallas_call(
    _final_kernel,
    grid=(GRID_N,),
    in_specs=[
        pl.BlockSpec((4, BN, 128), lambda i: (0, i, 0)),
        pl.BlockSpec((BN, 1), lambda i: (i, 0)),
        pl.BlockSpec((4, 128, 1024), lambda i: (0, 0, 0)),
        pl.BlockSpec((1, 1024), lambda i: (0, 0)),
        pl.BlockSpec((1024, 1024), lambda i: (0, 0)),
        pl.BlockSpec((1, 1024), lambda i: (0, 0)),
    ],
    out_specs=pl.BlockSpec((1, 1024), lambda i: (0, 0)),
    out_shape=jax.ShapeDtypeStruct((1, 1024), jnp.float32),
    scratch_shapes=[pltpu.VMEM((1, 1024), jnp.float32)],
    compiler_params=pltpu.CompilerParams(
        dimension_semantics=("arbitrary",)),
)

_agg1 = _make_agg(1, 16)
_agg2 = _make_agg(2, 32)
_agg3 = _make_agg(2, 64)
_agg4 = _make_agg(2, 128)
_agg5 = _make_agg(4, 128)
_gemm1 = _make_gemm(1, 16, 2, 32, sum_in=True)
_gemm2 = _make_gemm(2, 32, 2, 64)
_gemm3 = _make_gemm(2, 64, 2, 128)
_gemm4 = _make_gemm(2, 128, 4, 128)


def kernel(pos, edge_index, W1, b1, W2, b2, W3, b3, W4, b4, W5, b5, fcW, fcb):
    src = edge_index[0]
    dst = edge_index[1]
    pad = N + jnp.arange(EPAD - E, dtype=jnp.int32) % (NPAD - N)
    srcpad = jnp.concatenate([src, pad])
    dstpad = jnp.concatenate([dst, pad])
    srcp = srcpad.reshape(NS, NBLK, EB)
    dstp = dstpad.reshape(NS, NBLK, EB)
    srcw = srcpad.reshape(NS, NBLK_W, EB_W)
    dstw = dstpad.reshape(NS, NBLK_W, EB_W)
    pos_pad = jnp.pad(pos, ((0, NPAD - N), (0, 13)))
    zeros16 = jnp.zeros((NPAD, 16), jnp.float32)
    ones_rows = jnp.ones((EB, 16), jnp.float32)

    deg = _deg_kernel(zeros16, ones_rows, dstp)
    dinv, y0 = _prep(deg, pos_pad)

    a1 = _agg1(y0, zeros16, srcp, dstp)
    y1 = _gemm1(a1, dinv, jnp.pad(W1, ((0, 13), (0, 0)))[None], b1[None])
    a2 = _agg2(y1, zeros16, srcp, dstp)
    y2 = _gemm2(a2, dinv, W2.reshape(2, 32, 128), b2[None])
    a3 = _agg3(y2, zeros16, srcp, dstp)
    y3 = _gemm3(a3, dinv, W3.reshape(2, 64, 256), b3[None])
    a4 = _agg4(y3, zeros16, srcw, dstw)
    y4 = _gemm4(a4, dinv, W4.reshape(2, 128, 512), b4[None])
    a5 = _agg5(y4, zeros16, srcw, dstw)
    out = _final(a5, dinv, W5.reshape(4, 128, 1024), b5[None], fcW, fcb[None])
    return out.reshape(1024)
